# baseline (speedup 1.0000x reference)
import jax
import jax.numpy as jnp
from jax import lax
from jax.experimental import pallas as pl
from jax.experimental.pallas import tpu as pltpu

BS, SEQ, D, PLEN = 256, 196, 768, 10
OUT_SEQ = PLEN + SEQ
XROW = SEQ * D
PROW = PLEN * D
OROW = OUT_SEQ * D


def _x_copy(x_hbm, out_hbm, sem, i):
    return pltpu.make_async_copy(
        x_hbm.at[pl.ds(i * XROW, XROW)],
        out_hbm.at[pl.ds(i * OROW + PROW, XROW)],
        sem,
    )


def _p_copy(pool_hbm, out_hbm, sem, t, i):
    return pltpu.make_async_copy(
        pool_hbm.at[pl.ds(t * PROW, PROW)],
        out_hbm.at[pl.ds(i * OROW, PROW)],
        sem,
    )


def _body(tid_ref, x_hbm, pool_hbm, out_hbm, sem_x, sem_p):
    def issue(i, carry):
        _x_copy(x_hbm, out_hbm, sem_x, i).start()
        _p_copy(pool_hbm, out_hbm, sem_p, tid_ref[i], i).start()
        return carry

    lax.fori_loop(0, BS, issue, 0)

    def drain(i, carry):
        _x_copy(x_hbm, out_hbm, sem_x, i).wait()
        _p_copy(pool_hbm, out_hbm, sem_p, 0, i).wait()
        return carry

    lax.fori_loop(0, BS, drain, 0)


def kernel(x, prompt_pool, task_id):
    out_flat = pl.pallas_call(
        _body,
        in_specs=[
            pl.BlockSpec(memory_space=pltpu.SMEM),
            pl.BlockSpec(memory_space=pltpu.MemorySpace.HBM),
            pl.BlockSpec(memory_space=pltpu.MemorySpace.HBM),
        ],
        out_specs=pl.BlockSpec(memory_space=pltpu.MemorySpace.HBM),
        out_shape=jax.ShapeDtypeStruct((BS * OROW,), x.dtype),
        scratch_shapes=[pltpu.SemaphoreType.DMA, pltpu.SemaphoreType.DMA],
    )(task_id.astype(jnp.int32), x.reshape(-1), prompt_pool.reshape(-1))
    return out_flat.reshape(BS, OUT_SEQ, D)


# speedup vs baseline: 4.8473x; 4.8473x over previous
import jax
import jax.numpy as jnp
from jax.experimental import pallas as pl
from jax.experimental.pallas import tpu as pltpu

BS, SEQ, D, PLEN = 256, 196, 768, 10
OUT_SEQ = PLEN + SEQ
XROW = SEQ * D
PROW = PLEN * D
OROW = OUT_SEQ * D


def _body(tid_ref, x_ref, pool_ref, out_ref):
    out_ref[0, 0, :PROW] = pool_ref[0, 0]
    out_ref[0, 0, PROW:] = x_ref[0, 0]


def kernel(x, prompt_pool, task_id):
    grid_spec = pltpu.PrefetchScalarGridSpec(
        num_scalar_prefetch=1,
        grid=(BS,),
        in_specs=[
            pl.BlockSpec((1, 1, XROW), lambda i, tid: (i, 0, 0)),
            pl.BlockSpec((1, 1, PROW), lambda i, tid: (tid[i], 0, 0)),
        ],
        out_specs=pl.BlockSpec((1, 1, OROW), lambda i, tid: (i, 0, 0)),
    )
    out_flat = pl.pallas_call(
        _body,
        grid_spec=grid_spec,
        out_shape=jax.ShapeDtypeStruct((BS, 1, OROW), x.dtype),
    )(task_id.astype(jnp.int32), x.reshape(BS, 1, XROW),
      prompt_pool.reshape(-1, 1, PROW))
    return out_flat.reshape(BS, OUT_SEQ, D)


# speedup vs baseline: 8.4476x; 1.7428x over previous
import jax
import jax.numpy as jnp
from jax import lax
from jax.experimental import pallas as pl
from jax.experimental.pallas import tpu as pltpu
from jax.experimental.pallas import tpu_sc as plsc

BS, SEQ, D, PLEN, SESSIONS = 256, 196, 768, 10, 10
OUT_SEQ = PLEN + SEQ
XROW = SEQ * D
PROW = PLEN * D
OROW = OUT_SEQ * D
NC, NS = 2, 16
NW = NC * NS
SPW = BS // NW
NCHUNK = 12
CH = XROW // NCHUNK
NBUF = 4
TOT = SPW * NCHUNK


def _sc_body(x_hbm, pool_hbm, tid_hbm, out_hbm, idx_v, pv, bufs,
             sem_g, sem_in, sem_out):
    wid = lax.axis_index("s") * NC + lax.axis_index("c")
    base = pl.multiple_of(wid * SPW, SPW)

    pltpu.sync_copy(tid_hbm.at[pl.ds(base, SPW)], idx_v)
    gather = pltpu.make_async_copy(pool_hbm.at[idx_v], pv, sem_g)
    gather.start()

    def in_copy(c, b):
        j, p = c // NCHUNK, c % NCHUNK
        off = pl.multiple_of((base + j) * XROW + p * CH, 128)
        return pltpu.make_async_copy(
            x_hbm.at[pl.ds(off, CH)], bufs.at[b], sem_in.at[b])

    def out_copy(c, b):
        j, p = c // NCHUNK, c % NCHUNK
        off = pl.multiple_of((base + j) * OROW + PROW + p * CH, 128)
        return pltpu.make_async_copy(
            bufs.at[b], out_hbm.at[pl.ds(off, CH)], sem_out.at[b])

    for b in range(NBUF):
        in_copy(b, b).start()

    def group(g, carry):
        for b in range(NBUF):
            c = g * NBUF + b
            in_copy(c, b).wait()
            out_copy(c, b).start()
            out_copy(c, b).wait()
            in_copy(c + NBUF, b).start()
        return carry

    lax.fori_loop(0, TOT // NBUF - 1, group, 0)
    for b in range(NBUF):
        c = TOT - NBUF + b
        in_copy(c, b).wait()
        out_copy(c, b).start()
    for b in range(NBUF):
        out_copy(TOT - NBUF + b, b).wait()

    gather.wait()
    for j in range(SPW):
        off = pl.multiple_of((base + j) * OROW, 128)
        pltpu.sync_copy(pv.at[j], out_hbm.at[pl.ds(off, PROW)])


def kernel(x, prompt_pool, task_id):
    mesh = plsc.VectorSubcoreMesh(core_axis_name="c", subcore_axis_name="s")
    run = pl.kernel(
        _sc_body,
        out_type=jax.ShapeDtypeStruct((BS * OROW,), jnp.float32),
        mesh=mesh,
        scratch_types=[
            pltpu.VMEM((SPW,), jnp.int32),
            pltpu.VMEM((SPW, PROW), jnp.float32),
            pltpu.VMEM((NBUF, CH), jnp.float32),
            pltpu.SemaphoreType.DMA,
            pltpu.SemaphoreType.DMA((NBUF,)),
            pltpu.SemaphoreType.DMA((NBUF,)),
        ],
    )
    out_flat = run(x.reshape(-1), prompt_pool.reshape(SESSIONS, PROW),
                   task_id.astype(jnp.int32))
    return out_flat.reshape(BS, OUT_SEQ, D)
